# baseline (speedup 1.0000x reference)
import functools

import jax
import jax.numpy as jnp
from jax import lax
from jax.experimental import pallas as pl
from jax.experimental.pallas import tpu as pltpu
from jax.experimental.pallas import tpu_sc as plsc

N = 10000
K = 32
D = 128
H = 128
G = 16

NC = 2
NS = 16
NW = NC * NS
NPW = 320
NPAD = NW * NPW
CN = 4
NCH = NPW // CN
RPC = CN * K

BD = 1024
BT = 2048


def _dense_pre_body(x_ref, preWT, preb, WlT, WrT, hb, u_ref, s_ref):
    h0 = jnp.dot(x_ref[...], preWT[...], preferred_element_type=jnp.float32, precision=lax.Precision.HIGHEST)
    h0 = h0 + preb[...]
    u_ref[...] = jnp.dot(h0, WlT[...], preferred_element_type=jnp.float32, precision=lax.Precision.HIGHEST)
    s_ref[...] = (jnp.dot(h0, WrT[...], preferred_element_type=jnp.float32, precision=lax.Precision.HIGHEST)
                  + hb[...]) * (1.0 / K)


def _dense_mid_body(g_ref, WlT, WrT, hb, u_ref, s_ref):
    h1 = jnp.maximum(g_ref[...], 0.0)
    u_ref[...] = jnp.dot(h1, WlT[...], preferred_element_type=jnp.float32, precision=lax.Precision.HIGHEST)
    s_ref[...] = (jnp.dot(h1, WrT[...], preferred_element_type=jnp.float32, precision=lax.Precision.HIGHEST)
                  + hb[...]) * (1.0 / K)


def _dist_mlp_body(t_ref, w1a, b1a, w2a, b2a, d_ref):
    t = t_ref[0]
    acc = jnp.zeros((8, BT), jnp.float32)
    for g in range(H // 8):
        w1g = w1a[pl.ds(8 * g, 8), :]
        b1g = b1a[pl.ds(8 * g, 8), :]
        w2g = w2a[pl.ds(8 * g, 8), :]
        acc = acc + jnp.maximum(w1g * t + b1g, 0.0) * w2g
    d_ref[0] = (jnp.sum(acc, axis=0, keepdims=True)
                + b2a[0, 0]) * (1.0 / K)


def _pool_body(g_ref, batch_ref, predWc, predb, out_ref):
    p = jnp.dot(g_ref[...], predWc[...], preferred_element_type=jnp.float32, precision=lax.Precision.HIGHEST)
    b = batch_ref[...]
    oh = (b[None, :] == lax.broadcasted_iota(jnp.int32, (G, NPAD), 0)).astype(jnp.float32)
    sums = jnp.dot(oh, p, preferred_element_type=jnp.float32, precision=lax.Precision.HIGHEST)
    cnt = jnp.sum(oh, axis=1, keepdims=True)
    out_ref[...] = sums / jnp.maximum(cnt, 1.0) + predb[...]


def _wspec(shape):
    return pl.BlockSpec(shape, lambda *args: (0,) * len(shape))


def _dense_pre(xp, preWT, preb, WlT, WrT, hb):
    grid = (NPAD // BD,)
    return pl.pallas_call(
        _dense_pre_body,
        grid=grid,
        in_specs=[
            pl.BlockSpec((BD, D), lambda i: (i, 0)),
            _wspec((D, D)), _wspec((1, D)), _wspec((D, H)), _wspec((D, H)), _wspec((1, H)),
        ],
        out_specs=[pl.BlockSpec((BD, H), lambda i: (i, 0)),
                   pl.BlockSpec((BD, H), lambda i: (i, 0))],
        out_shape=[jax.ShapeDtypeStruct((NPAD, H), jnp.float32),
                   jax.ShapeDtypeStruct((NPAD, H), jnp.float32)],
    )(xp, preWT, preb, WlT, WrT, hb)


def _dense_mid(g1, WlT, WrT, hb):
    grid = (NPAD // BD,)
    return pl.pallas_call(
        _dense_mid_body,
        grid=grid,
        in_specs=[
            pl.BlockSpec((BD, H), lambda i: (i, 0)),
            _wspec((H, H)), _wspec((H, H)), _wspec((1, H)),
        ],
        out_specs=[pl.BlockSpec((BD, H), lambda i: (i, 0)),
                   pl.BlockSpec((BD, H), lambda i: (i, 0))],
        out_shape=[jax.ShapeDtypeStruct((NPAD, H), jnp.float32),
                   jax.ShapeDtypeStruct((NPAD, H), jnp.float32)],
    )(g1, WlT, WrT, hb)


def _dist_mlp(tmat, w1a, b1a, w2a, b2a):
    MW = NPW * K
    return pl.pallas_call(
        _dist_mlp_body,
        grid=(NW, MW // BT),
        in_specs=[
            pl.BlockSpec((1, 1, BT), lambda i, j: (i, 0, j)),
            _wspec((H, 1)), _wspec((H, 1)), _wspec((H, 1)), _wspec((1, 1)),
        ],
        out_specs=pl.BlockSpec((1, 1, BT), lambda i, j: (i, 0, j)),
        out_shape=jax.ShapeDtypeStruct((NW, 1, MW), jnp.float32),
    )(tmat, w1a, b1a, w2a, b2a)


def _pool(g2, bp, predWc, predb):
    return pl.pallas_call(
        _pool_body,
        out_shape=jax.ShapeDtypeStruct((G, 1), jnp.float32),
    )(g2, bp, predWc, predb)


def _make_sc_layer():
    mesh = plsc.VectorSubcoreMesh(core_axis_name="c", subcore_axis_name="s")

    @functools.partial(
        pl.kernel,
        mesh=mesh,
        out_type=jax.ShapeDtypeStruct((NW, NCH, CN, H), jnp.float32),
        scratch_types=[
            pltpu.VMEM((NCH, 128), jnp.int32),
            pltpu.VMEM((CN, K), jnp.float32),
            pltpu.VMEM((CN, K), jnp.float32),
            pltpu.VMEM((CN, H), jnp.float32),
            pltpu.VMEM((CN, H), jnp.float32),
            pltpu.VMEM((RPC, H), jnp.float32),
            pltpu.VMEM((RPC, H), jnp.float32),
            pltpu.VMEM((CN, H), jnp.float32),
            pltpu.VMEM((CN, H), jnp.float32),
            pltpu.VMEM_SHARED((NPAD, H), jnp.float32),
            pltpu.SemaphoreType.DMA,
            pltpu.SemaphoreType.DMA,
            pltpu.SemaphoreType.DMA,
            pltpu.SemaphoreType.DMA,
            pltpu.SemaphoreType.DMA,
            pltpu.SemaphoreType.DMA,
        ],
    )
    def sc_layer(table, idx, dw, sv, out,
                 idx_v, db0, db1, sb0, sb1, rows0, rows1, ov0, ov1, spm,
                 gsem0, gsem1, dsem0, dsem1, osem0, osem1):
        sid = lax.axis_index("s")
        wid = sid * NC + lax.axis_index("c")
        stripe = NPAD // NS
        pltpu.sync_copy(table.at[pl.ds(sid * stripe, stripe)],
                        spm.at[pl.ds(sid * stripe, stripe)])
        pltpu.sync_copy(idx.at[wid], idx_v)
        plsc.subcore_barrier()

        def start(ci, rows_v, db, sb, gsem, dsem):
            pltpu.async_copy(spm.at[idx_v.at[ci]], rows_v, gsem)
            pltpu.async_copy(dw.at[wid, ci], db, dsem)
            pltpu.async_copy(sv.at[wid, ci], sb, dsem)

        def wait(rows_v, db, sb, gsem, dsem):
            pltpu.make_async_copy(spm.at[idx_v.at[0]], rows_v, gsem).wait()
            pltpu.make_async_copy(dw.at[0, 0], db, dsem).wait()
            pltpu.make_async_copy(sv.at[0, 0], sb, dsem).wait()

        start(0, rows0, db0, sb0, gsem0, dsem0)
        start(1, rows1, db1, sb1, gsem1, dsem1)

        def compute_chunk(ci, rows_v, db, sb, ov, osem):
            def node(i, c2):
                svs = [sb[i, pl.ds(16 * j, 16)] for j in range(8)]
                dvecs = [db[i, pl.ds(16 * m, 16)] for m in range(K // 16)]
                accs = [jnp.zeros((16,), jnp.float32) for _ in range(8)]
                for k in range(K):
                    dsc = dvecs[k // 16][k % 16]
                    bb = jnp.full((16,), dsc, jnp.float32)
                    for j in range(8):
                        r = rows_v[i * K + k, pl.ds(16 * j, 16)]
                        accs[j] = accs[j] + jnp.maximum(bb * r + svs[j], 0.0)
                for j in range(8):
                    ov[i, pl.ds(16 * j, 16)] = accs[j]
                return c2

            lax.fori_loop(0, CN, node, 0)
            pltpu.async_copy(ov, out.at[wid, ci], osem)

        def super_chunk(c, carry):
            c0 = 2 * c
            for rows_v, db, sb, gsem, dsem, ov, osem, off in (
                    (rows0, db0, sb0, gsem0, dsem0, ov0, osem0, 0),
                    (rows1, db1, sb1, gsem1, dsem1, ov1, osem1, 1)):
                ci = c0 + off
                wait(rows_v, db, sb, gsem, dsem)

                @pl.when(c > 0)
                def _():
                    pltpu.make_async_copy(ov, out.at[0, 0], osem).wait()

                compute_chunk(ci, rows_v, db, sb, ov, osem)
                nxt = jnp.minimum(ci + 2, NCH - 1)
                start(nxt, rows_v, db, sb, gsem, dsem)
            return carry

        lax.fori_loop(0, NCH // 2, super_chunk, 0)
        wait(rows0, db0, sb0, gsem0, dsem0)
        wait(rows1, db1, sb1, gsem1, dsem1)
        pltpu.make_async_copy(ov0, out.at[0, 0], osem0).wait()
        pltpu.make_async_copy(ov1, out.at[0, 0], osem1).wait()

    return sc_layer


_sc_layer = _make_sc_layer()


def kernel(x, dists_max, dists_argmax, batch, pre_W, pre_b,
           c1_dW1, c1_db1, c1_dW2, c1_db2, c1_hW, c1_hb, c1_pW, c1_pb,
           c2_dW1, c2_db1, c2_dW2, c2_db2, c2_hW, c2_hb, c2_pW, c2_pb,
           pred_W, pred_b):
    pad = NPAD - N
    xp = x
    tp = jnp.pad(dists_max, ((0, pad), (0, 0)))
    ap = jnp.pad(dists_argmax.astype(jnp.int32), ((0, pad), (0, 0)))
    bp = jnp.pad(batch.astype(jnp.int32), (0, pad), constant_values=G)

    idx = ap.reshape(NW, NCH, 128)

    u1, s1 = _dense_pre(
        xp, pre_W.T, pre_b[None], c1_hW[:, :D].T, c1_hW[:, D:].T, c1_hb[None])

    tmat = tp.reshape(NW, 1, NPW * K)
    d1f = _dist_mlp(tmat, c1_dW1, c1_db1[:, None], c1_dW2.T, c1_db2[None, :])
    d2f = _dist_mlp(tmat, c2_dW1, c2_db1[:, None], c2_dW2.T, c2_db2[None, :])

    g1 = _sc_layer(u1, idx, d1f.reshape(NW, NCH, CN, K), s1.reshape(NW, NCH, CN, H))
    g1 = g1.reshape(NPAD, H)

    u2, s2 = _dense_mid(g1, c2_hW[:, :H].T, c2_hW[:, H:].T, c2_hb[None])

    g2 = _sc_layer(u2, idx, d2f.reshape(NW, NCH, CN, K), s2.reshape(NW, NCH, CN, H))
    g2 = g2.reshape(NPAD, H)

    return _pool(g2, bp, pred_W.T, pred_b[None])


# speedup vs baseline: 1.1646x; 1.1646x over previous
import functools

import jax
import jax.numpy as jnp
from jax import lax
from jax.experimental import pallas as pl
from jax.experimental.pallas import tpu as pltpu
from jax.experimental.pallas import tpu_sc as plsc

N = 10000
K = 32
D = 128
H = 128
G = 16

NC = 2
NS = 16
NW = NC * NS
NPW = 320
NPAD = NW * NPW
CN = 4
NCH = NPW // CN
RPC = CN * K

BD = 1024
BT = 2048


def _dense_pre_body(x_ref, preWT, preb, WlT, WrT, hb, u_ref, s_ref):
    h0 = jnp.dot(x_ref[...], preWT[...], preferred_element_type=jnp.float32, precision=lax.Precision.HIGHEST)
    h0 = h0 + preb[...]
    u_ref[...] = jnp.dot(h0, WlT[...], preferred_element_type=jnp.float32, precision=lax.Precision.HIGHEST)
    s_ref[...] = (jnp.dot(h0, WrT[...], preferred_element_type=jnp.float32, precision=lax.Precision.HIGHEST)
                  + hb[...]) * (1.0 / K)


def _dense_mid_body(g_ref, WlT, WrT, hb, u_ref, s_ref):
    h1 = jnp.maximum(g_ref[...], 0.0)
    u_ref[...] = jnp.dot(h1, WlT[...], preferred_element_type=jnp.float32, precision=lax.Precision.HIGHEST)
    s_ref[...] = (jnp.dot(h1, WrT[...], preferred_element_type=jnp.float32, precision=lax.Precision.HIGHEST)
                  + hb[...]) * (1.0 / K)


def _dist_mlp_body(t_ref, w1a, b1a, w2a, b2a, d_ref):
    t = t_ref[0]
    qa = jnp.maximum(w1a[...] * t + b1a[...], 0.0)
    d_ref[0] = (jnp.sum(qa * w2a[...], axis=0, keepdims=True)
                + b2a[0, 0]) * (1.0 / K)


def _pool_body(g_ref, batch_ref, predWc, predb, out_ref):
    p = jnp.dot(g_ref[...], predWc[...], preferred_element_type=jnp.float32, precision=lax.Precision.HIGHEST)
    b = batch_ref[...]
    oh = (b[None, :] == lax.broadcasted_iota(jnp.int32, (G, NPAD), 0)).astype(jnp.float32)
    sums = jnp.dot(oh, p, preferred_element_type=jnp.float32, precision=lax.Precision.HIGHEST)
    cnt = jnp.sum(oh, axis=1, keepdims=True)
    out_ref[...] = sums / jnp.maximum(cnt, 1.0) + predb[...]


def _wspec(shape):
    return pl.BlockSpec(shape, lambda *args: (0,) * len(shape))


def _dense_pre(xp, preWT, preb, WlT, WrT, hb):
    grid = (NPAD // BD,)
    return pl.pallas_call(
        _dense_pre_body,
        grid=grid,
        in_specs=[
            pl.BlockSpec((BD, D), lambda i: (i, 0)),
            _wspec((D, D)), _wspec((1, D)), _wspec((D, H)), _wspec((D, H)), _wspec((1, H)),
        ],
        out_specs=[pl.BlockSpec((BD, H), lambda i: (i, 0)),
                   pl.BlockSpec((BD, H), lambda i: (i, 0))],
        out_shape=[jax.ShapeDtypeStruct((NPAD, H), jnp.float32),
                   jax.ShapeDtypeStruct((NPAD, H), jnp.float32)],
    )(xp, preWT, preb, WlT, WrT, hb)


def _dense_mid(g1, WlT, WrT, hb):
    grid = (NPAD // BD,)
    return pl.pallas_call(
        _dense_mid_body,
        grid=grid,
        in_specs=[
            pl.BlockSpec((BD, H), lambda i: (i, 0)),
            _wspec((H, H)), _wspec((H, H)), _wspec((1, H)),
        ],
        out_specs=[pl.BlockSpec((BD, H), lambda i: (i, 0)),
                   pl.BlockSpec((BD, H), lambda i: (i, 0))],
        out_shape=[jax.ShapeDtypeStruct((NPAD, H), jnp.float32),
                   jax.ShapeDtypeStruct((NPAD, H), jnp.float32)],
    )(g1, WlT, WrT, hb)


def _dist_mlp(tmat, w1a, b1a, w2a, b2a):
    MW = NPW * K
    return pl.pallas_call(
        _dist_mlp_body,
        grid=(NW,),
        in_specs=[
            pl.BlockSpec((1, 1, MW), lambda i: (i, 0, 0)),
            _wspec((H, 1)), _wspec((H, 1)), _wspec((H, 1)), _wspec((1, 1)),
        ],
        out_specs=pl.BlockSpec((1, 1, MW), lambda i: (i, 0, 0)),
        out_shape=jax.ShapeDtypeStruct((NW, 1, MW), jnp.float32),
    )(tmat, w1a, b1a, w2a, b2a)


def _pool(g2, bp, predWc, predb):
    return pl.pallas_call(
        _pool_body,
        out_shape=jax.ShapeDtypeStruct((G, 1), jnp.float32),
    )(g2, bp, predWc, predb)


def _make_sc_layer():
    mesh = plsc.VectorSubcoreMesh(core_axis_name="c", subcore_axis_name="s")

    @functools.partial(
        pl.kernel,
        mesh=mesh,
        out_type=jax.ShapeDtypeStruct((NW, NCH, CN, H), jnp.float32),
        scratch_types=[
            pltpu.VMEM((NCH, 128), jnp.int32),
            pltpu.VMEM((CN, K), jnp.float32),
            pltpu.VMEM((CN, K), jnp.float32),
            pltpu.VMEM((CN, H), jnp.float32),
            pltpu.VMEM((CN, H), jnp.float32),
            pltpu.VMEM((RPC, H), jnp.float32),
            pltpu.VMEM((RPC, H), jnp.float32),
            pltpu.VMEM((CN, H), jnp.float32),
            pltpu.VMEM((CN, H), jnp.float32),
            pltpu.VMEM_SHARED((NPAD, H), jnp.float32),
            pltpu.SemaphoreType.DMA,
            pltpu.SemaphoreType.DMA,
            pltpu.SemaphoreType.DMA,
            pltpu.SemaphoreType.DMA,
            pltpu.SemaphoreType.DMA,
            pltpu.SemaphoreType.DMA,
        ],
    )
    def sc_layer(table, idx, dw, sv, out,
                 idx_v, db0, db1, sb0, sb1, rows0, rows1, ov0, ov1, spm,
                 gsem0, gsem1, dsem0, dsem1, osem0, osem1):
        sid = lax.axis_index("s")
        wid = sid * NC + lax.axis_index("c")
        stripe = NPAD // NS
        pltpu.sync_copy(table.at[pl.ds(sid * stripe, stripe)],
                        spm.at[pl.ds(sid * stripe, stripe)])
        pltpu.sync_copy(idx.at[wid], idx_v)
        plsc.subcore_barrier()

        def start(ci, rows_v, db, sb, gsem, dsem):
            pltpu.async_copy(spm.at[idx_v.at[ci]], rows_v, gsem)
            pltpu.async_copy(dw.at[wid, ci], db, dsem)
            pltpu.async_copy(sv.at[wid, ci], sb, dsem)

        def wait(rows_v, db, sb, gsem, dsem):
            pltpu.make_async_copy(spm.at[idx_v.at[0]], rows_v, gsem).wait()
            pltpu.make_async_copy(dw.at[0, 0], db, dsem).wait()
            pltpu.make_async_copy(sv.at[0, 0], sb, dsem).wait()

        start(0, rows0, db0, sb0, gsem0, dsem0)
        start(1, rows1, db1, sb1, gsem1, dsem1)

        def compute_chunk(ci, rows_v, db, sb, ov, osem):
            def node(i, c2):
                svs = [sb[i, pl.ds(16 * j, 16)] for j in range(8)]
                dvecs = [db[i, pl.ds(16 * m, 16)] for m in range(K // 16)]
                accs = [jnp.zeros((16,), jnp.float32) for _ in range(8)]
                for k in range(K):
                    dsc = dvecs[k // 16][k % 16]
                    bb = jnp.full((16,), dsc, jnp.float32)
                    for j in range(8):
                        r = rows_v[i * K + k, pl.ds(16 * j, 16)]
                        accs[j] = accs[j] + jnp.maximum(bb * r + svs[j], 0.0)
                for j in range(8):
                    ov[i, pl.ds(16 * j, 16)] = accs[j]
                return c2

            lax.fori_loop(0, CN, node, 0)
            pltpu.async_copy(ov, out.at[wid, ci], osem)

        def super_chunk(c, carry):
            c0 = 2 * c
            for rows_v, db, sb, gsem, dsem, ov, osem, off in (
                    (rows0, db0, sb0, gsem0, dsem0, ov0, osem0, 0),
                    (rows1, db1, sb1, gsem1, dsem1, ov1, osem1, 1)):
                ci = c0 + off
                wait(rows_v, db, sb, gsem, dsem)

                @pl.when(c > 0)
                def _():
                    pltpu.make_async_copy(ov, out.at[0, 0], osem).wait()

                compute_chunk(ci, rows_v, db, sb, ov, osem)
                nxt = jnp.minimum(ci + 2, NCH - 1)
                start(nxt, rows_v, db, sb, gsem, dsem)
            return carry

        lax.fori_loop(0, NCH // 2, super_chunk, 0)
        wait(rows0, db0, sb0, gsem0, dsem0)
        wait(rows1, db1, sb1, gsem1, dsem1)
        pltpu.make_async_copy(ov0, out.at[0, 0], osem0).wait()
        pltpu.make_async_copy(ov1, out.at[0, 0], osem1).wait()

    return sc_layer


_sc_layer = _make_sc_layer()


def kernel(x, dists_max, dists_argmax, batch, pre_W, pre_b,
           c1_dW1, c1_db1, c1_dW2, c1_db2, c1_hW, c1_hb, c1_pW, c1_pb,
           c2_dW1, c2_db1, c2_dW2, c2_db2, c2_hW, c2_hb, c2_pW, c2_pb,
           pred_W, pred_b):
    pad = NPAD - N
    xp = x
    tp = jnp.pad(dists_max, ((0, pad), (0, 0)))
    ap = jnp.pad(dists_argmax.astype(jnp.int32), ((0, pad), (0, 0)))
    bp = jnp.pad(batch.astype(jnp.int32), (0, pad), constant_values=G)

    idx = ap.reshape(NW, NCH, 128)

    u1, s1 = _dense_pre(
        xp, pre_W.T, pre_b[None], c1_hW[:, :D].T, c1_hW[:, D:].T, c1_hb[None])

    tmat = tp.reshape(NW, 1, NPW * K)
    d1f = _dist_mlp(tmat, c1_dW1, c1_db1[:, None], c1_dW2.T, c1_db2[None, :])
    d2f = _dist_mlp(tmat, c2_dW1, c2_db1[:, None], c2_dW2.T, c2_db2[None, :])

    g1 = _sc_layer(u1, idx, d1f.reshape(NW, NCH, CN, K), s1.reshape(NW, NCH, CN, H))
    g1 = g1.reshape(NPAD, H)

    u2, s2 = _dense_mid(g1, c2_hW[:, :H].T, c2_hW[:, H:].T, c2_hb[None])

    g2 = _sc_layer(u2, idx, d2f.reshape(NW, NCH, CN, K), s2.reshape(NW, NCH, CN, H))
    g2 = g2.reshape(NPAD, H)

    return _pool(g2, bp, pred_W.T, pred_b[None])


# speedup vs baseline: 1.1863x; 1.0187x over previous
import functools

import jax
import jax.numpy as jnp
from jax import lax
from jax.experimental import pallas as pl
from jax.experimental.pallas import tpu as pltpu
from jax.experimental.pallas import tpu_sc as plsc

N = 10000
K = 32
D = 128
H = 128
G = 16

NC = 2
NS = 16
NW = NC * NS
NPW = 320
NPAD = NW * NPW
CN = 4
NCH = NPW // CN
RPC = CN * K

BD = 1024
BT = 2048


def _fused_pre_body(x_ref, t_ref, preWT, preb, WlT, WrT, hb,
                    w1a, b1a, w2a, b2a, u_ref, s_ref, d_ref):
    h0 = jnp.dot(x_ref[...], preWT[...], preferred_element_type=jnp.float32, precision=lax.Precision.HIGHEST)
    h0 = h0 + preb[...]
    u_ref[...] = jnp.dot(h0, WlT[...], preferred_element_type=jnp.float32, precision=lax.Precision.HIGHEST)
    s_ref[...] = (jnp.dot(h0, WrT[...], preferred_element_type=jnp.float32, precision=lax.Precision.HIGHEST)
                  + hb[...]) * (1.0 / K)
    t = t_ref[0]
    qa = jnp.maximum(w1a[...] * t + b1a[...], 0.0)
    d_ref[0] = (jnp.sum(qa * w2a[...], axis=0, keepdims=True)
                + b2a[0, 0]) * (1.0 / K)


def _dense_mid_body(g_ref, WlT, WrT, hb, u_ref, s_ref):
    h1 = jnp.maximum(g_ref[...], 0.0)
    u_ref[...] = jnp.dot(h1, WlT[...], preferred_element_type=jnp.float32, precision=lax.Precision.HIGHEST)
    s_ref[...] = (jnp.dot(h1, WrT[...], preferred_element_type=jnp.float32, precision=lax.Precision.HIGHEST)
                  + hb[...]) * (1.0 / K)


def _dist_mlp_body(t_ref, w1a, b1a, w2a, b2a, d_ref):
    t = t_ref[0]
    qa = jnp.maximum(w1a[...] * t + b1a[...], 0.0)
    d_ref[0] = (jnp.sum(qa * w2a[...], axis=0, keepdims=True)
                + b2a[0, 0]) * (1.0 / K)


def _pool_body(g_ref, batch_ref, predWc, predb, out_ref):
    p = jnp.dot(g_ref[...], predWc[...], preferred_element_type=jnp.float32, precision=lax.Precision.HIGHEST)
    b = batch_ref[...]
    oh = (b[None, :] == lax.broadcasted_iota(jnp.int32, (G, NPAD), 0)).astype(jnp.float32)
    sums = jnp.dot(oh, p, preferred_element_type=jnp.float32, precision=lax.Precision.HIGHEST)
    cnt = jnp.sum(oh, axis=1, keepdims=True)
    out_ref[...] = sums / jnp.maximum(cnt, 1.0) + predb[...]


def _wspec(shape):
    return pl.BlockSpec(shape, lambda *args: (0,) * len(shape))


def _fused_pre(xp, tflat3, preWT, preb, WlT, WrT, hb, w1a, b1a, w2a, b2a):
    MW = NPW * K
    return pl.pallas_call(
        _fused_pre_body,
        grid=(NW,),
        in_specs=[
            pl.BlockSpec((NPW, D), lambda i: (i, 0)),
            pl.BlockSpec((1, 1, MW), lambda i: (0, 0, i)),
            _wspec((D, D)), _wspec((1, D)), _wspec((D, H)), _wspec((D, H)), _wspec((1, H)),
            _wspec((H, 1)), _wspec((H, 1)), _wspec((H, 1)), _wspec((1, 1)),
        ],
        out_specs=[pl.BlockSpec((NPW, H), lambda i: (i, 0)),
                   pl.BlockSpec((NPW, H), lambda i: (i, 0)),
                   pl.BlockSpec((1, 1, MW), lambda i: (i, 0, 0))],
        out_shape=[jax.ShapeDtypeStruct((NPAD, H), jnp.float32),
                   jax.ShapeDtypeStruct((NPAD, H), jnp.float32),
                   jax.ShapeDtypeStruct((NW, 1, MW), jnp.float32)],
    )(xp, tflat3, preWT, preb, WlT, WrT, hb, w1a, b1a, w2a, b2a)


def _dense_mid(g1, WlT, WrT, hb):
    grid = (NPAD // BD,)
    return pl.pallas_call(
        _dense_mid_body,
        grid=grid,
        in_specs=[
            pl.BlockSpec((BD, H), lambda i: (i, 0)),
            _wspec((H, H)), _wspec((H, H)), _wspec((1, H)),
        ],
        out_specs=[pl.BlockSpec((BD, H), lambda i: (i, 0)),
                   pl.BlockSpec((BD, H), lambda i: (i, 0))],
        out_shape=[jax.ShapeDtypeStruct((NPAD, H), jnp.float32),
                   jax.ShapeDtypeStruct((NPAD, H), jnp.float32)],
    )(g1, WlT, WrT, hb)


def _dist_mlp(tmat, w1a, b1a, w2a, b2a):
    MW = NPW * K
    return pl.pallas_call(
        _dist_mlp_body,
        grid=(NW,),
        in_specs=[
            pl.BlockSpec((1, 1, MW), lambda i: (0, 0, i)),
            _wspec((H, 1)), _wspec((H, 1)), _wspec((H, 1)), _wspec((1, 1)),
        ],
        out_specs=pl.BlockSpec((1, 1, MW), lambda i: (i, 0, 0)),
        out_shape=jax.ShapeDtypeStruct((NW, 1, MW), jnp.float32),
    )(tmat, w1a, b1a, w2a, b2a)


def _pool(g2, bp, predWc, predb):
    return pl.pallas_call(
        _pool_body,
        out_shape=jax.ShapeDtypeStruct((G, 1), jnp.float32),
    )(g2, bp, predWc, predb)


def _make_sc_layer():
    mesh = plsc.VectorSubcoreMesh(core_axis_name="c", subcore_axis_name="s")

    @functools.partial(
        pl.kernel,
        mesh=mesh,
        out_type=jax.ShapeDtypeStruct((NW, NCH, CN, H), jnp.float32),
        scratch_types=[
            pltpu.VMEM((NCH, 128), jnp.int32),
            pltpu.VMEM((CN, K), jnp.float32),
            pltpu.VMEM((CN, K), jnp.float32),
            pltpu.VMEM((CN, H), jnp.float32),
            pltpu.VMEM((CN, H), jnp.float32),
            pltpu.VMEM((RPC, H), jnp.float32),
            pltpu.VMEM((RPC, H), jnp.float32),
            pltpu.VMEM((CN, H), jnp.float32),
            pltpu.VMEM((CN, H), jnp.float32),
            pltpu.VMEM_SHARED((NPAD, H), jnp.float32),
            pltpu.SemaphoreType.DMA,
            pltpu.SemaphoreType.DMA,
            pltpu.SemaphoreType.DMA,
            pltpu.SemaphoreType.DMA,
            pltpu.SemaphoreType.DMA,
            pltpu.SemaphoreType.DMA,
        ],
    )
    def sc_layer(table, idx, dw, sv, out,
                 idx_v, db0, db1, sb0, sb1, rows0, rows1, ov0, ov1, spm,
                 gsem0, gsem1, dsem0, dsem1, osem0, osem1):
        sid = lax.axis_index("s")
        wid = sid * NC + lax.axis_index("c")
        stripe = NPAD // NS
        pltpu.sync_copy(table.at[pl.ds(sid * stripe, stripe)],
                        spm.at[pl.ds(sid * stripe, stripe)])
        pltpu.sync_copy(idx.at[wid], idx_v)
        plsc.subcore_barrier()

        def start(ci, rows_v, db, sb, gsem, dsem):
            pltpu.async_copy(spm.at[idx_v.at[ci]], rows_v, gsem)
            pltpu.async_copy(dw.at[wid, ci], db, dsem)
            pltpu.async_copy(sv.at[wid, ci], sb, dsem)

        def wait(rows_v, db, sb, gsem, dsem):
            pltpu.make_async_copy(spm.at[idx_v.at[0]], rows_v, gsem).wait()
            pltpu.make_async_copy(dw.at[0, 0], db, dsem).wait()
            pltpu.make_async_copy(sv.at[0, 0], sb, dsem).wait()

        start(0, rows0, db0, sb0, gsem0, dsem0)
        start(1, rows1, db1, sb1, gsem1, dsem1)

        def compute_chunk(ci, rows_v, db, sb, ov, osem):
            def node(i, c2):
                svs = [sb[i, pl.ds(16 * j, 16)] for j in range(8)]
                dvecs = [db[i, pl.ds(16 * m, 16)] for m in range(K // 16)]
                accs = [jnp.zeros((16,), jnp.float32) for _ in range(8)]
                for k in range(K):
                    dsc = dvecs[k // 16][k % 16]
                    bb = jnp.full((16,), dsc, jnp.float32)
                    for j in range(8):
                        r = rows_v[i * K + k, pl.ds(16 * j, 16)]
                        accs[j] = accs[j] + jnp.maximum(bb * r + svs[j], 0.0)
                for j in range(8):
                    ov[i, pl.ds(16 * j, 16)] = accs[j]
                return c2

            lax.fori_loop(0, CN, node, 0)
            pltpu.async_copy(ov, out.at[wid, ci], osem)

        def super_chunk(c, carry):
            c0 = 2 * c
            for rows_v, db, sb, gsem, dsem, ov, osem, off in (
                    (rows0, db0, sb0, gsem0, dsem0, ov0, osem0, 0),
                    (rows1, db1, sb1, gsem1, dsem1, ov1, osem1, 1)):
                ci = c0 + off
                wait(rows_v, db, sb, gsem, dsem)

                @pl.when(c > 0)
                def _():
                    pltpu.make_async_copy(ov, out.at[0, 0], osem).wait()

                compute_chunk(ci, rows_v, db, sb, ov, osem)
                nxt = jnp.minimum(ci + 2, NCH - 1)
                start(nxt, rows_v, db, sb, gsem, dsem)
            return carry

        lax.fori_loop(0, NCH // 2, super_chunk, 0)
        wait(rows0, db0, sb0, gsem0, dsem0)
        wait(rows1, db1, sb1, gsem1, dsem1)
        pltpu.make_async_copy(ov0, out.at[0, 0], osem0).wait()
        pltpu.make_async_copy(ov1, out.at[0, 0], osem1).wait()

    return sc_layer


_sc_layer = _make_sc_layer()


def kernel(x, dists_max, dists_argmax, batch, pre_W, pre_b,
           c1_dW1, c1_db1, c1_dW2, c1_db2, c1_hW, c1_hb, c1_pW, c1_pb,
           c2_dW1, c2_db1, c2_dW2, c2_db2, c2_hW, c2_hb, c2_pW, c2_pb,
           pred_W, pred_b):
    pad = NPAD - N
    ap = jnp.pad(dists_argmax.astype(jnp.int32), ((0, pad), (0, 0)))
    bp = jnp.pad(batch.astype(jnp.int32), (0, pad), constant_values=G)

    idx = ap.reshape(NW, NCH, 128)

    tflat3 = dists_max.reshape(1, 1, N * K)
    u1, s1, d1f = _fused_pre(
        x, tflat3, pre_W.T, pre_b[None], c1_hW[:, :D].T, c1_hW[:, D:].T,
        c1_hb[None], c1_dW1, c1_db1[:, None], c1_dW2.T, c1_db2[None, :])

    d2f = _dist_mlp(tflat3, c2_dW1, c2_db1[:, None], c2_dW2.T, c2_db2[None, :])

    g1 = _sc_layer(u1, idx, d1f.reshape(NW, NCH, CN, K), s1.reshape(NW, NCH, CN, H))
    g1 = g1.reshape(NPAD, H)

    u2, s2 = _dense_mid(g1, c2_hW[:, :H].T, c2_hW[:, H:].T, c2_hb[None])

    g2 = _sc_layer(u2, idx, d2f.reshape(NW, NCH, CN, K), s2.reshape(NW, NCH, CN, H))
    g2 = g2.reshape(NPAD, H)

    return _pool(g2, bp, pred_W.T, pred_b[None])


# speedup vs baseline: 1.1927x; 1.0053x over previous
import functools

import jax
import jax.numpy as jnp
from jax import lax
from jax.experimental import pallas as pl
from jax.experimental.pallas import tpu as pltpu
from jax.experimental.pallas import tpu_sc as plsc

N = 10000
K = 32
D = 128
H = 128
G = 16

NC = 2
NS = 16
NW = NC * NS
NPW = 320
NPAD = NW * NPW
CN = 4
NCH = NPW // CN
RPC = CN * K

BD = 1024
BT = 2048


def _fused_pre_body(x_ref, t_ref, preWT, preb, WlT, WrT, hb,
                    w1a, b1a, w2a, b2a, u_ref, s_ref, d_ref):
    h0 = jnp.dot(x_ref[...], preWT[...], preferred_element_type=jnp.float32, precision=lax.Precision.HIGHEST)
    h0 = h0 + preb[...]
    u_ref[...] = jnp.dot(h0, WlT[...], preferred_element_type=jnp.float32, precision=lax.Precision.HIGHEST)
    s_ref[...] = (jnp.dot(h0, WrT[...], preferred_element_type=jnp.float32, precision=lax.Precision.HIGHEST)
                  + hb[...]) * (1.0 / K)
    t = t_ref[0]
    qa = jnp.maximum(w1a[...] * t + b1a[...], 0.0)
    d_ref[0] = (jnp.sum(qa * w2a[...], axis=0, keepdims=True)
                + b2a[0, 0]) * (1.0 / K)


def _dense_mid_body(g_ref, WlT, WrT, hb, u_ref, s_ref):
    h1 = jnp.maximum(g_ref[...], 0.0)
    u_ref[...] = jnp.dot(h1, WlT[...], preferred_element_type=jnp.float32, precision=lax.Precision.HIGHEST)
    s_ref[...] = (jnp.dot(h1, WrT[...], preferred_element_type=jnp.float32, precision=lax.Precision.HIGHEST)
                  + hb[...]) * (1.0 / K)


def _dist_mlp_body(t_ref, w1a, b1a, w2a, b2a, d_ref):
    t = t_ref[0]
    qa = jnp.maximum(w1a[...] * t + b1a[...], 0.0)
    d_ref[0] = (jnp.sum(qa * w2a[...], axis=0, keepdims=True)
                + b2a[0, 0]) * (1.0 / K)


def _pool_body(g_ref, batch_ref, predWc, predb, out_ref):
    p = jnp.dot(g_ref[...], predWc[...], preferred_element_type=jnp.float32, precision=lax.Precision.HIGHEST)
    b = batch_ref[...]
    oh = (b[None, :] == lax.broadcasted_iota(jnp.int32, (G, NPAD), 0)).astype(jnp.float32)
    sums = jnp.dot(oh, p, preferred_element_type=jnp.float32, precision=lax.Precision.HIGHEST)
    cnt = jnp.sum(oh, axis=1, keepdims=True)
    out_ref[...] = sums / jnp.maximum(cnt, 1.0) + predb[...]


def _wspec(shape):
    return pl.BlockSpec(shape, lambda *args: (0,) * len(shape))


def _fused_pre(xp, tflat3, preWT, preb, WlT, WrT, hb, w1a, b1a, w2a, b2a):
    MW = NPW * K
    return pl.pallas_call(
        _fused_pre_body,
        grid=(NW,),
        in_specs=[
            pl.BlockSpec((NPW, D), lambda i: (i, 0)),
            pl.BlockSpec((1, 1, MW), lambda i: (0, 0, i)),
            _wspec((D, D)), _wspec((1, D)), _wspec((D, H)), _wspec((D, H)), _wspec((1, H)),
            _wspec((H, 1)), _wspec((H, 1)), _wspec((H, 1)), _wspec((1, 1)),
        ],
        out_specs=[pl.BlockSpec((NPW, H), lambda i: (i, 0)),
                   pl.BlockSpec((NPW, H), lambda i: (i, 0)),
                   pl.BlockSpec((1, 1, MW), lambda i: (i, 0, 0))],
        out_shape=[jax.ShapeDtypeStruct((NPAD, H), jnp.float32),
                   jax.ShapeDtypeStruct((NPAD, H), jnp.float32),
                   jax.ShapeDtypeStruct((NW, 1, MW), jnp.float32)],
    )(xp, tflat3, preWT, preb, WlT, WrT, hb, w1a, b1a, w2a, b2a)


def _dense_mid(g1, WlT, WrT, hb):
    grid = (NPAD // BD,)
    return pl.pallas_call(
        _dense_mid_body,
        grid=grid,
        in_specs=[
            pl.BlockSpec((BD, H), lambda i: (i, 0)),
            _wspec((H, H)), _wspec((H, H)), _wspec((1, H)),
        ],
        out_specs=[pl.BlockSpec((BD, H), lambda i: (i, 0)),
                   pl.BlockSpec((BD, H), lambda i: (i, 0))],
        out_shape=[jax.ShapeDtypeStruct((NPAD, H), jnp.float32),
                   jax.ShapeDtypeStruct((NPAD, H), jnp.float32)],
    )(g1, WlT, WrT, hb)


def _dist_mlp(tmat, w1a, b1a, w2a, b2a):
    MW = NPW * K
    return pl.pallas_call(
        _dist_mlp_body,
        grid=(NW,),
        in_specs=[
            pl.BlockSpec((1, 1, MW), lambda i: (0, 0, i)),
            _wspec((H, 1)), _wspec((H, 1)), _wspec((H, 1)), _wspec((1, 1)),
        ],
        out_specs=pl.BlockSpec((1, 1, MW), lambda i: (i, 0, 0)),
        out_shape=jax.ShapeDtypeStruct((NW, 1, MW), jnp.float32),
    )(tmat, w1a, b1a, w2a, b2a)


def _pool(g2, bp, predWc, predb):
    return pl.pallas_call(
        _pool_body,
        out_shape=jax.ShapeDtypeStruct((G, 1), jnp.float32),
    )(g2, bp, predWc, predb)


def _make_sc_layer():
    mesh = plsc.VectorSubcoreMesh(core_axis_name="c", subcore_axis_name="s")

    @functools.partial(
        pl.kernel,
        mesh=mesh,
        out_type=jax.ShapeDtypeStruct((NPAD, H), jnp.float32),
        scratch_types=[
            pltpu.VMEM((NCH, 128), jnp.int32),
            pltpu.VMEM((CN, K), jnp.float32),
            pltpu.VMEM((CN, K), jnp.float32),
            pltpu.VMEM((CN, H), jnp.float32),
            pltpu.VMEM((CN, H), jnp.float32),
            pltpu.VMEM((RPC, H), jnp.float32),
            pltpu.VMEM((RPC, H), jnp.float32),
            pltpu.VMEM((CN, H), jnp.float32),
            pltpu.VMEM((CN, H), jnp.float32),
            pltpu.VMEM_SHARED((NPAD, H), jnp.float32),
            pltpu.SemaphoreType.DMA,
            pltpu.SemaphoreType.DMA,
            pltpu.SemaphoreType.DMA,
            pltpu.SemaphoreType.DMA,
            pltpu.SemaphoreType.DMA,
            pltpu.SemaphoreType.DMA,
        ],
    )
    def sc_layer(table, idx, dw, sv, out,
                 idx_v, db0, db1, sb0, sb1, rows0, rows1, ov0, ov1, spm,
                 gsem0, gsem1, dsem0, dsem1, osem0, osem1):
        sid = lax.axis_index("s")
        wid = sid * NC + lax.axis_index("c")
        stripe = NPAD // NS
        pltpu.sync_copy(table.at[pl.ds(sid * stripe, stripe)],
                        spm.at[pl.ds(sid * stripe, stripe)])
        pltpu.sync_copy(idx.at[wid], idx_v)
        plsc.subcore_barrier()

        def start(ci, rows_v, db, sb, gsem, dsem):
            base = wid * NPW + ci * CN
            pltpu.async_copy(spm.at[idx_v.at[ci]], rows_v, gsem)
            pltpu.async_copy(dw.at[pl.ds(base, CN)], db, dsem)
            pltpu.async_copy(sv.at[pl.ds(base, CN)], sb, dsem)

        def wait(rows_v, db, sb, gsem, dsem):
            pltpu.make_async_copy(spm.at[idx_v.at[0]], rows_v, gsem).wait()
            pltpu.make_async_copy(dw.at[pl.ds(0, CN)], db, dsem).wait()
            pltpu.make_async_copy(sv.at[pl.ds(0, CN)], sb, dsem).wait()

        start(0, rows0, db0, sb0, gsem0, dsem0)
        start(1, rows1, db1, sb1, gsem1, dsem1)

        def compute_chunk(ci, rows_v, db, sb, ov, osem):
            def node(i, c2):
                svs = [sb[i, pl.ds(16 * j, 16)] for j in range(8)]
                dvecs = [db[i, pl.ds(16 * m, 16)] for m in range(K // 16)]
                accs = [jnp.zeros((16,), jnp.float32) for _ in range(8)]
                for k in range(K):
                    dsc = dvecs[k // 16][k % 16]
                    bb = jnp.full((16,), dsc, jnp.float32)
                    for j in range(8):
                        r = rows_v[i * K + k, pl.ds(16 * j, 16)]
                        accs[j] = accs[j] + jnp.maximum(bb * r + svs[j], 0.0)
                for j in range(8):
                    ov[i, pl.ds(16 * j, 16)] = accs[j]
                return c2

            lax.fori_loop(0, CN, node, 0)
            pltpu.async_copy(ov, out.at[pl.ds(wid * NPW + ci * CN, CN)], osem)

        def super_chunk(c, carry):
            c0 = 2 * c
            for rows_v, db, sb, gsem, dsem, ov, osem, off in (
                    (rows0, db0, sb0, gsem0, dsem0, ov0, osem0, 0),
                    (rows1, db1, sb1, gsem1, dsem1, ov1, osem1, 1)):
                ci = c0 + off
                wait(rows_v, db, sb, gsem, dsem)

                @pl.when(c > 0)
                def _():
                    pltpu.make_async_copy(ov, out.at[pl.ds(0, CN)], osem).wait()

                compute_chunk(ci, rows_v, db, sb, ov, osem)
                nxt = jnp.minimum(ci + 2, NCH - 1)
                start(nxt, rows_v, db, sb, gsem, dsem)
            return carry

        lax.fori_loop(0, NCH // 2, super_chunk, 0)
        wait(rows0, db0, sb0, gsem0, dsem0)
        wait(rows1, db1, sb1, gsem1, dsem1)
        pltpu.make_async_copy(ov0, out.at[pl.ds(0, CN)], osem0).wait()
        pltpu.make_async_copy(ov1, out.at[pl.ds(0, CN)], osem1).wait()

    return sc_layer


_sc_layer = _make_sc_layer()


def kernel(x, dists_max, dists_argmax, batch, pre_W, pre_b,
           c1_dW1, c1_db1, c1_dW2, c1_db2, c1_hW, c1_hb, c1_pW, c1_pb,
           c2_dW1, c2_db1, c2_dW2, c2_db2, c2_hW, c2_hb, c2_pW, c2_pb,
           pred_W, pred_b):
    pad = NPAD - N
    ap = jnp.pad(dists_argmax.astype(jnp.int32), ((0, pad), (0, 0)))
    bp = jnp.pad(batch.astype(jnp.int32), (0, pad), constant_values=G)

    idx = ap.reshape(NW, NCH, 128)

    tflat3 = dists_max.reshape(1, 1, N * K)
    u1, s1, d1f = _fused_pre(
        x, tflat3, pre_W.T, pre_b[None], c1_hW[:, :D].T, c1_hW[:, D:].T,
        c1_hb[None], c1_dW1, c1_db1[:, None], c1_dW2.T, c1_db2[None, :])

    d2f = _dist_mlp(tflat3, c2_dW1, c2_db1[:, None], c2_dW2.T, c2_db2[None, :])

    g1 = _sc_layer(u1, idx, d1f.reshape(NPAD, K), s1)

    u2, s2 = _dense_mid(g1, c2_hW[:, :H].T, c2_hW[:, H:].T, c2_hb[None])

    g2 = _sc_layer(u2, idx, d2f.reshape(NPAD, K), s2)

    return _pool(g2, bp, pred_W.T, pred_b[None])
